# baseline (speedup 1.0000x reference)
import functools

import jax
import jax.numpy as jnp
from jax import lax
from jax.experimental import pallas as pl
from jax.experimental.pallas import tpu as pltpu
from jax.experimental.pallas import tpu_sc as plsc

_VOCAB_A, _DIM_A = 1000000, 32
_VOCAB_B, _DIM_B = 100000, 64
_BATCH, _HIST = 16384, 50

_L = 128
_NW = 32
_BPW = _BATCH // _NW
_KB = _BPW // _L
_BLOCKS = _HIST * _KB
_TR_A = _DIM_A // 8
_TR_B = _DIM_B // 8


def _transpose_block(rows_v, t_v, dim):
    lanes = jnp.arange(16, dtype=jnp.int32)
    for d in range(dim):
        for j in range(_L // 16):
            v = plsc.load_gather(rows_v, [lanes + 16 * j,
                                          jnp.full((16,), d, jnp.int32)])
            t_v[d // 8, d % 8, pl.ds(16 * j, 16)] = v


def _body(idx_a_hbm, idx_b_hbm, tab_a_hbm, tab_b_hbm, out_a_hbm, out_b_hbm,
          idxa_v, idxb_v, ra0, ra1, ta0, ta1, rb0, rb1, tb0, tb1,
          sem_g, sem_st):
    nc = plsc.get_sparse_core_info().num_cores
    wid = lax.axis_index("s") * nc + lax.axis_index("c")
    b0 = wid * _BPW

    pltpu.sync_copy(idx_a_hbm.at[:, pl.ds(b0, _BPW)], idxa_v)
    pltpu.sync_copy(idx_b_hbm.at[:, pl.ds(b0, _BPW)], idxb_v)

    def run(tab_hbm, out_hbm, idx_v, rows, ts, dim):
        def dst(g):
            h, kb = g // _KB, g % _KB
            return out_hbm.at[h, :, wid * _KB + kb]

        def fire(g, buf):
            h, kb = g // _KB, g % _KB
            return pltpu.async_copy(
                tab_hbm.at[idx_v.at[h, pl.ds(kb * _L, _L)]], buf, sem_g)

        def step(t, carry):
            g0 = 2 * t
            cps = [fire(g0, rows[0]), fire(g0 + 1, rows[1])]
            for k in range(2):
                cps[k].wait()

                @pl.when(t > 0)
                def _drain():
                    pltpu.make_async_copy(ts[k], dst(g0 + k), sem_st).wait()

                _transpose_block(rows[k], ts[k], dim)
                pltpu.async_copy(ts[k], dst(g0 + k), sem_st)
            return carry

        lax.fori_loop(0, _BLOCKS // 2, step, 0)
        for k in range(2):
            pltpu.make_async_copy(ts[k], dst(_BLOCKS - 2 + k), sem_st).wait()

    run(tab_a_hbm, out_a_hbm, idxa_v, (ra0, ra1), (ta0, ta1), _DIM_A)
    run(tab_b_hbm, out_b_hbm, idxb_v, (rb0, rb1), (tb0, tb1), _DIM_B)


@jax.jit
def _lookup(dom_a_idx, dom_b_idx, table_a, table_b):
    mesh = plsc.VectorSubcoreMesh(core_axis_name="c", subcore_axis_name="s")
    k = pl.kernel(
        _body,
        out_type=(
            jax.ShapeDtypeStruct((_HIST, _TR_A, _BATCH // _L, 8, _L),
                                 jnp.float32),
            jax.ShapeDtypeStruct((_HIST, _TR_B, _BATCH // _L, 8, _L),
                                 jnp.float32),
        ),
        mesh=mesh,
        scratch_types=[
            pltpu.VMEM((_HIST, _BPW), jnp.int32),
            pltpu.VMEM((_HIST, _BPW), jnp.int32),
            pltpu.VMEM((_L, _DIM_A), jnp.float32),
            pltpu.VMEM((_L, _DIM_A), jnp.float32),
            pltpu.VMEM((_TR_A, 8, _L), jnp.float32),
            pltpu.VMEM((_TR_A, 8, _L), jnp.float32),
            pltpu.VMEM((_L, _DIM_B), jnp.float32),
            pltpu.VMEM((_L, _DIM_B), jnp.float32),
            pltpu.VMEM((_TR_B, 8, _L), jnp.float32),
            pltpu.VMEM((_TR_B, 8, _L), jnp.float32),
            pltpu.SemaphoreType.DMA,
            pltpu.SemaphoreType.DMA,
        ],
        compiler_params=pltpu.CompilerParams(use_tc_tiling_on_sc=False,
                                             needs_layout_passes=False),
    )
    ia = jnp.transpose(dom_a_idx)
    ib = jnp.transpose(dom_b_idx)
    ka, kb = k(ia, ib, table_a, table_b)

    def finish(arr, dim):
        p = arr.transpose(0, 1, 3, 2, 4).reshape(_HIST, dim, _BATCH)
        return p.transpose(2, 0, 1)

    return finish(ka, _DIM_A), finish(kb, _DIM_B)


def kernel(dom_a_idx, dom_b_idx, table_a, table_b):
    return _lookup(dom_a_idx, dom_b_idx, table_a, table_b)


# speedup vs baseline: 2.1287x; 2.1287x over previous
import functools

import jax
import jax.numpy as jnp
from jax import lax
from jax.experimental import pallas as pl
from jax.experimental.pallas import tpu as pltpu
from jax.experimental.pallas import tpu_sc as plsc

_VOCAB_A, _DIM_A = 1000000, 32
_VOCAB_B, _DIM_B = 100000, 64
_BATCH, _HIST = 16384, 50

_L = 128
_NW = 32
_BPW = _BATCH // _NW
_KB = _BPW // _L
_BLOCKS = _HIST * _KB
_TR_A = _DIM_A // 8
_TR_B = _DIM_B // 8


def _transpose_block(rows_v, t_v, dim):
    iota = jnp.arange(16, dtype=jnp.int32)

    def krot(k, carry):
        rot = jnp.bitwise_and(iota + k, 15)
        for d0 in range(0, dim, 16):
            d_vec = rot + d0
            tr_vec = jnp.right_shift(d_vec, 3)
            s_vec = jnp.bitwise_and(d_vec, 7)
            for j0 in range(0, _L, 16):
                lanes = iota + j0
                v = plsc.load_gather(rows_v, [lanes, d_vec])
                plsc.store_scatter(t_v, [tr_vec, s_vec, lanes], v)
        return carry

    lax.fori_loop(0, 16, krot, 0)


def _body(idx_a_hbm, idx_b_hbm, tab_a_hbm, tab_b_hbm, out_a_hbm, out_b_hbm,
          idxa_v, idxb_v, ra0, ra1, ta0, ta1, rb0, rb1, tb0, tb1,
          sem_g, sem_st):
    nc = plsc.get_sparse_core_info().num_cores
    wid = lax.axis_index("s") * nc + lax.axis_index("c")
    b0 = wid * _BPW

    pltpu.sync_copy(idx_a_hbm.at[:, pl.ds(b0, _BPW)], idxa_v)
    pltpu.sync_copy(idx_b_hbm.at[:, pl.ds(b0, _BPW)], idxb_v)

    def run(tab_hbm, out_hbm, idx_v, rows, ts, dim):
        def dst(g):
            h, kb = g // _KB, g % _KB
            return out_hbm.at[h, :, wid * _KB + kb]

        def fire(g, buf):
            h, kb = g // _KB, g % _KB
            return pltpu.async_copy(
                tab_hbm.at[idx_v.at[h, pl.ds(kb * _L, _L)]], buf, sem_g)

        def step(t, carry):
            g0 = 2 * t
            cps = [fire(g0, rows[0]), fire(g0 + 1, rows[1])]
            for k in range(2):
                cps[k].wait()

                @pl.when(t > 0)
                def _drain():
                    pltpu.make_async_copy(ts[k], dst(g0 + k), sem_st).wait()

                _transpose_block(rows[k], ts[k], dim)
                pltpu.async_copy(ts[k], dst(g0 + k), sem_st)
            return carry

        lax.fori_loop(0, _BLOCKS // 2, step, 0)
        for k in range(2):
            pltpu.make_async_copy(ts[k], dst(_BLOCKS - 2 + k), sem_st).wait()

    run(tab_a_hbm, out_a_hbm, idxa_v, (ra0, ra1), (ta0, ta1), _DIM_A)
    run(tab_b_hbm, out_b_hbm, idxb_v, (rb0, rb1), (tb0, tb1), _DIM_B)


@jax.jit
def _lookup(dom_a_idx, dom_b_idx, table_a, table_b):
    mesh = plsc.VectorSubcoreMesh(core_axis_name="c", subcore_axis_name="s")
    k = pl.kernel(
        _body,
        out_type=(
            jax.ShapeDtypeStruct((_HIST, _TR_A, _BATCH // _L, 8, _L),
                                 jnp.float32),
            jax.ShapeDtypeStruct((_HIST, _TR_B, _BATCH // _L, 8, _L),
                                 jnp.float32),
        ),
        mesh=mesh,
        scratch_types=[
            pltpu.VMEM((_HIST, _BPW), jnp.int32),
            pltpu.VMEM((_HIST, _BPW), jnp.int32),
            pltpu.VMEM((_L, _DIM_A), jnp.float32),
            pltpu.VMEM((_L, _DIM_A), jnp.float32),
            pltpu.VMEM((_TR_A, 8, _L), jnp.float32),
            pltpu.VMEM((_TR_A, 8, _L), jnp.float32),
            pltpu.VMEM((_L, _DIM_B), jnp.float32),
            pltpu.VMEM((_L, _DIM_B), jnp.float32),
            pltpu.VMEM((_TR_B, 8, _L), jnp.float32),
            pltpu.VMEM((_TR_B, 8, _L), jnp.float32),
            pltpu.SemaphoreType.DMA,
            pltpu.SemaphoreType.DMA,
        ],
        compiler_params=pltpu.CompilerParams(use_tc_tiling_on_sc=False,
                                             needs_layout_passes=False),
    )
    ia = jnp.transpose(dom_a_idx)
    ib = jnp.transpose(dom_b_idx)

    def repack(tab):
        rows, d = tab.shape
        packed = tab.reshape(rows * d // 128, 128)
        packed = jax.lax.optimization_barrier(packed)
        return packed.reshape(rows, d)

    ka, kb = k(ia, ib, repack(table_a), repack(table_b))

    def finish(arr, dim):
        p = arr.transpose(0, 1, 3, 2, 4).reshape(_HIST, dim, _BATCH)
        return p.transpose(2, 0, 1)

    return finish(ka, _DIM_A), finish(kb, _DIM_B)


def kernel(dom_a_idx, dom_b_idx, table_a, table_b):
    return _lookup(dom_a_idx, dom_b_idx, table_a, table_b)


# speedup vs baseline: 2.2189x; 1.0424x over previous
import functools

import jax
import jax.numpy as jnp
from jax import lax
from jax.experimental import pallas as pl
from jax.experimental.pallas import tpu as pltpu
from jax.experimental.pallas import tpu_sc as plsc

_VOCAB_A, _DIM_A = 1000000, 32
_VOCAB_B, _DIM_B = 100000, 64
_BATCH, _HIST = 16384, 50

_L = 128
_NW = 32
_BPW = _BATCH // _NW
_KB = _BPW // _L
_BLOCKS = _HIST * _KB
_TR_A = _DIM_A // 8
_TR_B = _DIM_B // 8


def _transpose_block(rows_v, t_v, dim):
    iota = jnp.arange(16, dtype=jnp.int32)

    def krot(kk, carry):
        for k4 in range(4):
            rot = jnp.bitwise_and(iota + (4 * kk + k4), 15)
            for d0 in range(0, dim, 16):
                d_vec = rot + d0
                tr_vec = jnp.right_shift(d_vec, 3)
                s_vec = jnp.bitwise_and(d_vec, 7)
                for j0 in range(0, _L, 16):
                    lanes = iota + j0
                    v = plsc.load_gather(rows_v, [lanes, d_vec])
                    plsc.store_scatter(t_v, [tr_vec, s_vec, lanes], v)
        return carry

    lax.fori_loop(0, 4, krot, 0)


def _body(idx_a_hbm, idx_b_hbm, tab_a_hbm, tab_b_hbm, out_a_hbm, out_b_hbm,
          idxa_v, idxb_v, ra0, ra1, ta0, ta1, rb0, rb1, tb0, tb1,
          sem_g, sem_st):
    nc = plsc.get_sparse_core_info().num_cores
    wid = lax.axis_index("s") * nc + lax.axis_index("c")
    b0 = wid * _BPW

    pltpu.sync_copy(idx_a_hbm.at[:, pl.ds(b0, _BPW)], idxa_v)
    pltpu.sync_copy(idx_b_hbm.at[:, pl.ds(b0, _BPW)], idxb_v)

    def run(tab_hbm, out_hbm, idx_v, rows, ts, dim):
        def dst(g):
            h, kb = g // _KB, g % _KB
            return out_hbm.at[h, :, wid * _KB + kb]

        def fire(g, buf):
            h, kb = g // _KB, g % _KB
            return pltpu.async_copy(
                tab_hbm.at[idx_v.at[h, pl.ds(kb * _L, _L)]], buf, sem_g)

        def step(t, carry):
            g0 = 2 * t
            cps = [fire(g0, rows[0]), fire(g0 + 1, rows[1])]
            for k in range(2):
                cps[k].wait()

                @pl.when(t > 0)
                def _drain():
                    pltpu.make_async_copy(ts[k], dst(g0 + k), sem_st).wait()

                _transpose_block(rows[k], ts[k], dim)
                pltpu.async_copy(ts[k], dst(g0 + k), sem_st)
            return carry

        lax.fori_loop(0, _BLOCKS // 2, step, 0)
        for k in range(2):
            pltpu.make_async_copy(ts[k], dst(_BLOCKS - 2 + k), sem_st).wait()

    run(tab_a_hbm, out_a_hbm, idxa_v, (ra0, ra1), (ta0, ta1), _DIM_A)
    run(tab_b_hbm, out_b_hbm, idxb_v, (rb0, rb1), (tb0, tb1), _DIM_B)


@jax.jit
def _lookup(dom_a_idx, dom_b_idx, table_a, table_b):
    mesh = plsc.VectorSubcoreMesh(core_axis_name="c", subcore_axis_name="s")
    k = pl.kernel(
        _body,
        out_type=(
            jax.ShapeDtypeStruct((_HIST, _TR_A, _BATCH // _L, 8, _L),
                                 jnp.float32),
            jax.ShapeDtypeStruct((_HIST, _TR_B, _BATCH // _L, 8, _L),
                                 jnp.float32),
        ),
        mesh=mesh,
        scratch_types=[
            pltpu.VMEM((_HIST, _BPW), jnp.int32),
            pltpu.VMEM((_HIST, _BPW), jnp.int32),
            pltpu.VMEM((_L, _DIM_A), jnp.float32),
            pltpu.VMEM((_L, _DIM_A), jnp.float32),
            pltpu.VMEM((_TR_A, 8, _L), jnp.float32),
            pltpu.VMEM((_TR_A, 8, _L), jnp.float32),
            pltpu.VMEM((_L, _DIM_B), jnp.float32),
            pltpu.VMEM((_L, _DIM_B), jnp.float32),
            pltpu.VMEM((_TR_B, 8, _L), jnp.float32),
            pltpu.VMEM((_TR_B, 8, _L), jnp.float32),
            pltpu.SemaphoreType.DMA,
            pltpu.SemaphoreType.DMA,
        ],
        compiler_params=pltpu.CompilerParams(use_tc_tiling_on_sc=False,
                                             needs_layout_passes=False),
    )
    ia = jnp.transpose(dom_a_idx)
    ib = jnp.transpose(dom_b_idx)

    def repack(tab):
        rows, d = tab.shape
        packed = tab.reshape(rows * d // 128, 128)
        packed = jax.lax.optimization_barrier(packed)
        return packed.reshape(rows, d)

    ka, kb = k(ia, ib, repack(table_a), repack(table_b))

    def finish(arr, dim):
        p = arr.transpose(0, 1, 3, 2, 4).reshape(_HIST, dim, _BATCH)
        return p.transpose(2, 0, 1)

    return finish(ka, _DIM_A), finish(kb, _DIM_B)


def kernel(dom_a_idx, dom_b_idx, table_a, table_b):
    return _lookup(dom_a_idx, dom_b_idx, table_a, table_b)


# speedup vs baseline: 2.5538x; 1.1509x over previous
import functools

import jax
import jax.numpy as jnp
from jax import lax
from jax.experimental import pallas as pl
from jax.experimental.pallas import tpu as pltpu
from jax.experimental.pallas import tpu_sc as plsc

_VOCAB_A, _DIM_A = 1000000, 32
_VOCAB_B, _DIM_B = 100000, 64
_BATCH, _HIST = 16384, 50

_L = 128
_NW = 32
_BPW = _BATCH // _NW
_KB = _BPW // _L
_BLOCKS = _HIST * _KB
_TR_A = _DIM_A // 8
_TR_B = _DIM_B // 8


def _transpose_block(rows_v, t_v, dim):
    iota = jnp.arange(16, dtype=jnp.int32)

    @plsc.parallel_loop(0, 16, step=1, unroll=4)
    def _krot(k):
        rot = jnp.bitwise_and(iota + k, 15)
        for d0 in range(0, dim, 16):
            d_vec = rot + d0
            tr_vec = jnp.right_shift(d_vec, 3)
            s_vec = jnp.bitwise_and(d_vec, 7)
            for j0 in range(0, _L, 16):
                lanes = iota + j0
                v = plsc.load_gather(rows_v, [lanes, d_vec])
                plsc.store_scatter(t_v, [tr_vec, s_vec, lanes], v)


def _body(idx_a_hbm, idx_b_hbm, tab_a_hbm, tab_b_hbm, out_a_hbm, out_b_hbm,
          idxa_v, idxb_v, ra0, ra1, ta0, ta1, rb0, rb1, tb0, tb1,
          sem_g, sem_st):
    nc = plsc.get_sparse_core_info().num_cores
    wid = lax.axis_index("s") * nc + lax.axis_index("c")
    b0 = wid * _BPW

    pltpu.sync_copy(idx_a_hbm.at[:, pl.ds(b0, _BPW)], idxa_v)
    pltpu.sync_copy(idx_b_hbm.at[:, pl.ds(b0, _BPW)], idxb_v)

    def run(tab_hbm, out_hbm, idx_v, rows, ts, dim):
        def dst(g):
            h, kb = g // _KB, g % _KB
            return out_hbm.at[h, :, wid * _KB + kb]

        def fire(g, buf):
            h, kb = g // _KB, g % _KB
            return pltpu.async_copy(
                tab_hbm.at[idx_v.at[h, pl.ds(kb * _L, _L)]], buf, sem_g)

        def step(t, carry):
            g0 = 2 * t
            cps = [fire(g0, rows[0]), fire(g0 + 1, rows[1])]
            for k in range(2):
                cps[k].wait()

                @pl.when(t > 0)
                def _drain():
                    pltpu.make_async_copy(ts[k], dst(g0 + k), sem_st).wait()

                _transpose_block(rows[k], ts[k], dim)
                pltpu.async_copy(ts[k], dst(g0 + k), sem_st)
            return carry

        lax.fori_loop(0, _BLOCKS // 2, step, 0)
        for k in range(2):
            pltpu.make_async_copy(ts[k], dst(_BLOCKS - 2 + k), sem_st).wait()

    run(tab_a_hbm, out_a_hbm, idxa_v, (ra0, ra1), (ta0, ta1), _DIM_A)
    run(tab_b_hbm, out_b_hbm, idxb_v, (rb0, rb1), (tb0, tb1), _DIM_B)


@jax.jit
def _lookup(dom_a_idx, dom_b_idx, table_a, table_b):
    mesh = plsc.VectorSubcoreMesh(core_axis_name="c", subcore_axis_name="s")
    k = pl.kernel(
        _body,
        out_type=(
            jax.ShapeDtypeStruct((_HIST, _TR_A, _BATCH // _L, 8, _L),
                                 jnp.float32),
            jax.ShapeDtypeStruct((_HIST, _TR_B, _BATCH // _L, 8, _L),
                                 jnp.float32),
        ),
        mesh=mesh,
        scratch_types=[
            pltpu.VMEM((_HIST, _BPW), jnp.int32),
            pltpu.VMEM((_HIST, _BPW), jnp.int32),
            pltpu.VMEM((_L, _DIM_A), jnp.float32),
            pltpu.VMEM((_L, _DIM_A), jnp.float32),
            pltpu.VMEM((_TR_A, 8, _L), jnp.float32),
            pltpu.VMEM((_TR_A, 8, _L), jnp.float32),
            pltpu.VMEM((_L, _DIM_B), jnp.float32),
            pltpu.VMEM((_L, _DIM_B), jnp.float32),
            pltpu.VMEM((_TR_B, 8, _L), jnp.float32),
            pltpu.VMEM((_TR_B, 8, _L), jnp.float32),
            pltpu.SemaphoreType.DMA,
            pltpu.SemaphoreType.DMA,
        ],
        compiler_params=pltpu.CompilerParams(use_tc_tiling_on_sc=False,
                                             needs_layout_passes=False),
    )
    ia = jnp.transpose(dom_a_idx)
    ib = jnp.transpose(dom_b_idx)

    def repack(tab):
        rows, d = tab.shape
        packed = tab.reshape(rows * d // 128, 128)
        packed = jax.lax.optimization_barrier(packed)
        return packed.reshape(rows, d)

    ka, kb = k(ia, ib, repack(table_a), repack(table_b))

    def finish(arr, dim):
        p = arr.transpose(0, 1, 3, 2, 4).reshape(_HIST, dim, _BATCH)
        return p.transpose(2, 0, 1)

    return finish(ka, _DIM_A), finish(kb, _DIM_B)


def kernel(dom_a_idx, dom_b_idx, table_a, table_b):
    return _lookup(dom_a_idx, dom_b_idx, table_a, table_b)


# speedup vs baseline: 3.5616x; 1.3946x over previous
import functools

import jax
import jax.numpy as jnp
from jax import lax
from jax.experimental import pallas as pl
from jax.experimental.pallas import tpu as pltpu
from jax.experimental.pallas import tpu_sc as plsc

_VOCAB_A, _DIM_A = 1000000, 32
_VOCAB_B, _DIM_B = 100000, 64
_BATCH, _HIST = 16384, 50

_L = 128
_NW = 32
_BPW = _BATCH // _NW
_KB = _BPW // _L
_BLOCKS = _HIST * _KB
_TR_A = _DIM_A // 8
_TR_B = _DIM_B // 8


def _transpose_block(rows_v, t_v, dim):
    iota = jnp.arange(16, dtype=jnp.int32)

    @plsc.parallel_loop(0, 16, step=1, unroll=4)
    def _krot(k):
        rot = jnp.bitwise_and(iota + k, 15)
        for d0 in range(0, dim, 16):
            d_vec = rot + d0
            tr_vec = jnp.right_shift(d_vec, 3)
            s_vec = jnp.bitwise_and(d_vec, 7)
            for j0 in range(0, _L, 16):
                lanes = iota + j0
                v = plsc.load_gather(rows_v, [lanes, d_vec])
                plsc.store_scatter(t_v, [tr_vec, s_vec, lanes], v)


def _diag_transpose_packed(buf, outb, rows, cols, out_row0):
    iota = jnp.arange(16, dtype=jnp.int32)
    pack = 128 // rows

    @plsc.parallel_loop(0, 16, step=1, unroll=4)
    def _krot(k):
        rot = jnp.bitwise_and(iota + k, 15)
        for d0 in range(0, rows, 16):
            d_vec = rot + d0
            for l0 in range(0, cols, 16):
                lanes = iota + l0
                q_vec = jnp.right_shift(lanes, pack // 2) + out_row0
                z_vec = jnp.bitwise_and(lanes, pack - 1) * rows + d_vec
                v = plsc.load_gather(buf, [d_vec, lanes])
                plsc.store_scatter(outb, [q_vec, z_vec], v)


def _repack_body(ta_hbm, tb_hbm, tail_a_hbm, tail_b_hbm, pa_hbm, pb_hbm,
                 ba0, ba1, oa0, oa1, bb0, bb1, ob0, ob1, sem_g, sem_st):
    nc = plsc.get_sparse_core_info().num_cores
    wid = lax.axis_index("s") * nc + lax.axis_index("c")

    def runp(t_hbm, p_hbm, bufs, outs, rows, cw, npairs):
        rows_out = rows * cw // 128

        def src(q):
            return t_hbm.at[:, pl.ds(q * cw, cw)]

        def dst(q):
            return p_hbm.at[pl.ds(q * rows_out, rows_out)]

        def step(i, carry):
            q0 = jnp.minimum((2 * i) * _NW + wid, npairs - 1)
            q1 = jnp.minimum((2 * i + 1) * _NW + wid, npairs - 1)
            qq = (q0, q1)
            cps = [pltpu.async_copy(src(q0), bufs[0], sem_g),
                   pltpu.async_copy(src(q1), bufs[1], sem_g)]
            for k in range(2):
                cps[k].wait()

                @pl.when(i > 0)
                def _drain():
                    pltpu.make_async_copy(outs[k], dst(qq[k]), sem_st).wait()

                _diag_transpose_packed(bufs[k], outs[k], rows, cw, 0)
                pltpu.async_copy(outs[k], dst(qq[k]), sem_st)
            return carry

        niter = (npairs + 2 * _NW - 1) // (2 * _NW)
        lax.fori_loop(0, niter, step, 0)
        for k in range(2):
            pltpu.make_async_copy(outs[k], dst(jnp.int32(npairs - 1)),
                                  sem_st).wait()

    runp(ta_hbm, pa_hbm, (ba0, ba1), (oa0, oa1), 32, 256, 3906)
    runp(tb_hbm, pb_hbm, (bb0, bb1), (ob0, ob1), 64, 256, 390)

    @pl.when(wid == 0)
    def _tail_a():
        pltpu.sync_copy(tail_a_hbm, oa0.at[pl.ds(0, 16)])
        pltpu.sync_copy(oa0.at[pl.ds(0, 16)], pa_hbm.at[pl.ds(249984, 16)])

    @pl.when(wid == 1)
    def _tail_b():
        pltpu.sync_copy(tb_hbm.at[:, pl.ds(99840, 128)],
                        bb0.at[:, pl.ds(0, 128)])
        _diag_transpose_packed(bb0, ob0, 64, 128, 0)
        pltpu.sync_copy(ob0.at[pl.ds(0, 64)], pb_hbm.at[pl.ds(49920, 64)])

    @pl.when(wid == 2)
    def _tail_b2():
        pltpu.sync_copy(tail_b_hbm, ob1.at[pl.ds(0, 16)])
        pltpu.sync_copy(ob1.at[pl.ds(0, 16)], pb_hbm.at[pl.ds(49984, 16)])


def _body(idx_a_hbm, idx_b_hbm, tab_a_hbm, tab_b_hbm, out_a_hbm, out_b_hbm,
          idxa_v, idxb_v, ra0, ra1, ta0, ta1, rb0, rb1, tb0, tb1,
          sem_g, sem_st):
    nc = plsc.get_sparse_core_info().num_cores
    wid = lax.axis_index("s") * nc + lax.axis_index("c")
    b0 = wid * _BPW

    pltpu.sync_copy(idx_a_hbm.at[:, pl.ds(b0, _BPW)], idxa_v)
    pltpu.sync_copy(idx_b_hbm.at[:, pl.ds(b0, _BPW)], idxb_v)

    def run(tab_hbm, out_hbm, idx_v, rows, ts, dim):
        def dst(g):
            h, kb = g // _KB, g % _KB
            return out_hbm.at[h, :, wid * _KB + kb]

        def fire(g, buf):
            h, kb = g // _KB, g % _KB
            return pltpu.async_copy(
                tab_hbm.at[idx_v.at[h, pl.ds(kb * _L, _L)]], buf, sem_g)

        def step(t, carry):
            g0 = 2 * t
            cps = [fire(g0, rows[0]), fire(g0 + 1, rows[1])]
            for k in range(2):
                cps[k].wait()

                @pl.when(t > 0)
                def _drain():
                    pltpu.make_async_copy(ts[k], dst(g0 + k), sem_st).wait()

                _transpose_block(rows[k], ts[k], dim)
                pltpu.async_copy(ts[k], dst(g0 + k), sem_st)
            return carry

        lax.fori_loop(0, _BLOCKS // 2, step, 0)
        for k in range(2):
            pltpu.make_async_copy(ts[k], dst(_BLOCKS - 2 + k), sem_st).wait()

    run(tab_a_hbm, out_a_hbm, idxa_v, (ra0, ra1), (ta0, ta1), _DIM_A)
    run(tab_b_hbm, out_b_hbm, idxb_v, (rb0, rb1), (tb0, tb1), _DIM_B)


@jax.jit
def _lookup(dom_a_idx, dom_b_idx, table_a, table_b):
    mesh = plsc.VectorSubcoreMesh(core_axis_name="c", subcore_axis_name="s")
    rk = pl.kernel(
        _repack_body,
        out_type=(
            jax.ShapeDtypeStruct((_VOCAB_A * _DIM_A // 128, 128), jnp.float32),
            jax.ShapeDtypeStruct((_VOCAB_B * _DIM_B // 128, 128), jnp.float32),
        ),
        mesh=mesh,
        scratch_types=[
            pltpu.VMEM((32, 256), jnp.float32),
            pltpu.VMEM((32, 256), jnp.float32),
            pltpu.VMEM((64, 128), jnp.float32),
            pltpu.VMEM((64, 128), jnp.float32),
            pltpu.VMEM((64, 256), jnp.float32),
            pltpu.VMEM((64, 256), jnp.float32),
            pltpu.VMEM((128, 128), jnp.float32),
            pltpu.VMEM((128, 128), jnp.float32),
            pltpu.SemaphoreType.DMA,
            pltpu.SemaphoreType.DMA,
        ],
        compiler_params=pltpu.CompilerParams(use_tc_tiling_on_sc=True,
                                             needs_layout_passes=False),
    )
    k = pl.kernel(
        _body,
        out_type=(
            jax.ShapeDtypeStruct((_HIST, _TR_A, _BATCH // _L, 8, _L),
                                 jnp.float32),
            jax.ShapeDtypeStruct((_HIST, _TR_B, _BATCH // _L, 8, _L),
                                 jnp.float32),
        ),
        mesh=mesh,
        scratch_types=[
            pltpu.VMEM((_HIST, _BPW), jnp.int32),
            pltpu.VMEM((_HIST, _BPW), jnp.int32),
            pltpu.VMEM((_L, _DIM_A), jnp.float32),
            pltpu.VMEM((_L, _DIM_A), jnp.float32),
            pltpu.VMEM((_TR_A, 8, _L), jnp.float32),
            pltpu.VMEM((_TR_A, 8, _L), jnp.float32),
            pltpu.VMEM((_L, _DIM_B), jnp.float32),
            pltpu.VMEM((_L, _DIM_B), jnp.float32),
            pltpu.VMEM((_TR_B, 8, _L), jnp.float32),
            pltpu.VMEM((_TR_B, 8, _L), jnp.float32),
            pltpu.SemaphoreType.DMA,
            pltpu.SemaphoreType.DMA,
        ],
        compiler_params=pltpu.CompilerParams(use_tc_tiling_on_sc=False,
                                             needs_layout_passes=False),
    )
    ia = jnp.transpose(dom_a_idx)
    ib = jnp.transpose(dom_b_idx)

    tail_a = table_a[_VOCAB_A - 64:].reshape(16, 128)
    tail_b = table_b[_VOCAB_B - 32:].reshape(16, 128)
    pa, pb = rk(jnp.transpose(table_a), jnp.transpose(table_b),
                tail_a, tail_b)
    ta = pa.reshape(_VOCAB_A, _DIM_A)
    tb = pb.reshape(_VOCAB_B, _DIM_B)

    ka, kb = k(ia, ib, ta, tb)

    def finish(arr, dim):
        p = arr.transpose(0, 1, 3, 2, 4).reshape(_HIST, dim, _BATCH)
        return p.transpose(2, 0, 1)

    return finish(ka, _DIM_A), finish(kb, _DIM_B)


def kernel(dom_a_idx, dom_b_idx, table_a, table_b):
    return _lookup(dom_a_idx, dom_b_idx, table_a, table_b)


# speedup vs baseline: 4.1352x; 1.1611x over previous
import functools

import jax
import jax.numpy as jnp
from jax import lax
from jax.experimental import pallas as pl
from jax.experimental.pallas import tpu as pltpu
from jax.experimental.pallas import tpu_sc as plsc

_VOCAB_A, _DIM_A = 1000000, 32
_VOCAB_B, _DIM_B = 100000, 64
_BATCH, _HIST = 16384, 50

_L = 128
_NW = 32
_BPW = _BATCH // _NW
_KB = _BPW // _L
_BLOCKS = _HIST * _KB
_TR_A = _DIM_A // 8
_TR_B = _DIM_B // 8


def _transpose_block(rows_v, t_v, dim):
    iota = jnp.arange(16, dtype=jnp.int32)

    @plsc.parallel_loop(0, 16, step=1, unroll=4)
    def _krot(k):
        rot = jnp.bitwise_and(iota + k, 15)
        for d0 in range(0, dim, 16):
            d_vec = rot + d0
            tr_vec = jnp.right_shift(d_vec, 3)
            s_vec = jnp.bitwise_and(d_vec, 7)
            for j0 in range(0, _L, 16):
                lanes = iota + j0
                v = plsc.load_gather(rows_v, [lanes, d_vec])
                plsc.store_scatter(t_v, [tr_vec, s_vec, lanes], v)


def _diag_transpose_packed(buf, outb, rows, cols, out_row0):
    iota = jnp.arange(16, dtype=jnp.int32)
    pack = 128 // rows

    @plsc.parallel_loop(0, 16, step=1, unroll=4)
    def _krot(k):
        rot = jnp.bitwise_and(iota + k, 15)
        for d0 in range(0, rows, 16):
            d_vec = rot + d0
            for l0 in range(0, cols, 16):
                lanes = iota + l0
                q_vec = jnp.right_shift(lanes, pack // 2) + out_row0
                z_vec = jnp.bitwise_and(lanes, pack - 1) * rows + d_vec
                v = plsc.load_gather(buf, [d_vec, lanes])
                plsc.store_scatter(outb, [q_vec, z_vec], v)


def _repack_body(ta_hbm, tb_hbm, tail_a_hbm, tail_b_hbm, pa_hbm, pb_hbm,
                 ba0, ba1, oa0, oa1, bb0, bb1, ob0, ob1, sem_g, sem_st):
    nc = plsc.get_sparse_core_info().num_cores
    wid = lax.axis_index("s") * nc + lax.axis_index("c")

    def runp(t_hbm, p_hbm, bufs, outs, rows, cw, npairs):
        rows_out = rows * cw // 128

        def src(q):
            return t_hbm.at[:, pl.ds(q * cw, cw)]

        def dst(q):
            return p_hbm.at[pl.ds(q * rows_out, rows_out)]

        def step(i, carry):
            q0 = jnp.minimum((2 * i) * _NW + wid, npairs - 1)
            q1 = jnp.minimum((2 * i + 1) * _NW + wid, npairs - 1)
            qq = (q0, q1)
            cps = [pltpu.async_copy(src(q0), bufs[0], sem_g),
                   pltpu.async_copy(src(q1), bufs[1], sem_g)]
            for k in range(2):
                cps[k].wait()

                @pl.when(i > 0)
                def _drain():
                    pltpu.make_async_copy(outs[k], dst(qq[k]), sem_st).wait()

                _diag_transpose_packed(bufs[k], outs[k], rows, cw, 0)
                pltpu.async_copy(outs[k], dst(qq[k]), sem_st)
            return carry

        niter = (npairs + 2 * _NW - 1) // (2 * _NW)
        lax.fori_loop(0, niter, step, 0)
        for k in range(2):
            pltpu.make_async_copy(outs[k], dst(jnp.int32(npairs - 1)),
                                  sem_st).wait()

    runp(ta_hbm, pa_hbm, (ba0, ba1), (oa0, oa1), 32, 256, 3906)
    runp(tb_hbm, pb_hbm, (bb0, bb1), (ob0, ob1), 64, 256, 390)

    @pl.when(wid == 0)
    def _tail_a():
        pltpu.sync_copy(tail_a_hbm, oa0.at[pl.ds(0, 16)])
        pltpu.sync_copy(oa0.at[pl.ds(0, 16)], pa_hbm.at[pl.ds(249984, 16)])

    @pl.when(wid == 1)
    def _tail_b():
        pltpu.sync_copy(tb_hbm.at[:, pl.ds(99840, 128)],
                        bb0.at[:, pl.ds(0, 128)])
        _diag_transpose_packed(bb0, ob0, 64, 128, 0)
        pltpu.sync_copy(ob0.at[pl.ds(0, 64)], pb_hbm.at[pl.ds(49920, 64)])

    @pl.when(wid == 2)
    def _tail_b2():
        pltpu.sync_copy(tail_b_hbm, ob1.at[pl.ds(0, 16)])
        pltpu.sync_copy(ob1.at[pl.ds(0, 16)], pb_hbm.at[pl.ds(49984, 16)])


def _body(idx_a_hbm, idx_b_hbm, tab_a_hbm, tab_b_hbm, out_a_hbm, out_b_hbm,
          idxa_v, idxb_v, ra0, ra1, ta0, ta1, rb0, rb1, tb0, tb1,
          sem_g, sem_st):
    nc = plsc.get_sparse_core_info().num_cores
    wid = lax.axis_index("s") * nc + lax.axis_index("c")
    b0 = wid * _BPW

    pltpu.sync_copy(idx_a_hbm.at[:, pl.ds(b0, _BPW)], idxa_v)
    pltpu.sync_copy(idx_b_hbm.at[:, pl.ds(b0, _BPW)], idxb_v)

    def run(tab_hbm, out_hbm, idx_v, rows, ts, dim):
        def dst(g):
            h, kb = g // _KB, g % _KB
            return out_hbm.at[h, :, wid * _KB + kb]

        def gather_copy(g, buf):
            g = jnp.minimum(g, _BLOCKS - 1)
            h, kb = g // _KB, g % _KB
            return pltpu.make_async_copy(
                tab_hbm.at[idx_v.at[h, pl.ds(kb * _L, _L)]], buf, sem_g)

        gather_copy(0, rows[0]).start()
        gather_copy(1, rows[1]).start()

        def step(t, carry):
            g0 = 2 * t
            for k in range(2):
                gather_copy(g0 + k, rows[k]).wait()

                @pl.when(t > 0)
                def _drain():
                    pltpu.make_async_copy(ts[k], dst(g0 + k), sem_st).wait()

                _transpose_block(rows[k], ts[k], dim)
                pltpu.async_copy(ts[k], dst(g0 + k), sem_st)
                gather_copy(g0 + 2 + k, rows[k]).start()
            return carry

        lax.fori_loop(0, _BLOCKS // 2, step, 0)
        for k in range(2):
            gather_copy(_BLOCKS - 1, rows[k]).wait()
            pltpu.make_async_copy(ts[k], dst(_BLOCKS - 2 + k), sem_st).wait()

    run(tab_a_hbm, out_a_hbm, idxa_v, (ra0, ra1), (ta0, ta1), _DIM_A)
    run(tab_b_hbm, out_b_hbm, idxb_v, (rb0, rb1), (tb0, tb1), _DIM_B)


@jax.jit
def _lookup(dom_a_idx, dom_b_idx, table_a, table_b):
    mesh = plsc.VectorSubcoreMesh(core_axis_name="c", subcore_axis_name="s")
    rk = pl.kernel(
        _repack_body,
        out_type=(
            jax.ShapeDtypeStruct((_VOCAB_A * _DIM_A // 128, 128), jnp.float32),
            jax.ShapeDtypeStruct((_VOCAB_B * _DIM_B // 128, 128), jnp.float32),
        ),
        mesh=mesh,
        scratch_types=[
            pltpu.VMEM((32, 256), jnp.float32),
            pltpu.VMEM((32, 256), jnp.float32),
            pltpu.VMEM((64, 128), jnp.float32),
            pltpu.VMEM((64, 128), jnp.float32),
            pltpu.VMEM((64, 256), jnp.float32),
            pltpu.VMEM((64, 256), jnp.float32),
            pltpu.VMEM((128, 128), jnp.float32),
            pltpu.VMEM((128, 128), jnp.float32),
            pltpu.SemaphoreType.DMA,
            pltpu.SemaphoreType.DMA,
        ],
        compiler_params=pltpu.CompilerParams(use_tc_tiling_on_sc=True,
                                             needs_layout_passes=False),
    )
    k = pl.kernel(
        _body,
        out_type=(
            jax.ShapeDtypeStruct((_HIST, _TR_A, _BATCH // _L, 8, _L),
                                 jnp.float32),
            jax.ShapeDtypeStruct((_HIST, _TR_B, _BATCH // _L, 8, _L),
                                 jnp.float32),
        ),
        mesh=mesh,
        scratch_types=[
            pltpu.VMEM((_HIST, _BPW), jnp.int32),
            pltpu.VMEM((_HIST, _BPW), jnp.int32),
            pltpu.VMEM((_L, _DIM_A), jnp.float32),
            pltpu.VMEM((_L, _DIM_A), jnp.float32),
            pltpu.VMEM((_TR_A, 8, _L), jnp.float32),
            pltpu.VMEM((_TR_A, 8, _L), jnp.float32),
            pltpu.VMEM((_L, _DIM_B), jnp.float32),
            pltpu.VMEM((_L, _DIM_B), jnp.float32),
            pltpu.VMEM((_TR_B, 8, _L), jnp.float32),
            pltpu.VMEM((_TR_B, 8, _L), jnp.float32),
            pltpu.SemaphoreType.DMA,
            pltpu.SemaphoreType.DMA,
        ],
        compiler_params=pltpu.CompilerParams(use_tc_tiling_on_sc=False,
                                             needs_layout_passes=False),
    )
    ia = jnp.transpose(dom_a_idx)
    ib = jnp.transpose(dom_b_idx)

    tail_a = table_a[_VOCAB_A - 64:].reshape(16, 128)
    tail_b = table_b[_VOCAB_B - 32:].reshape(16, 128)
    pa, pb = rk(jnp.transpose(table_a), jnp.transpose(table_b),
                tail_a, tail_b)
    ta = pa.reshape(_VOCAB_A, _DIM_A)
    tb = pb.reshape(_VOCAB_B, _DIM_B)

    ka, kb = k(ia, ib, ta, tb)

    def finish(arr, dim):
        p = arr.transpose(0, 1, 3, 2, 4).reshape(_HIST, dim, _BATCH)
        return p.transpose(2, 0, 1)

    return finish(ka, _DIM_A), finish(kb, _DIM_B)


def kernel(dom_a_idx, dom_b_idx, table_a, table_b):
    return _lookup(dom_a_idx, dom_b_idx, table_a, table_b)


# speedup vs baseline: 5.3194x; 1.2864x over previous
import functools

import jax
import jax.numpy as jnp
from jax import lax
from jax.experimental import pallas as pl
from jax.experimental.pallas import tpu as pltpu
from jax.experimental.pallas import tpu_sc as plsc

_VOCAB_A, _DIM_A = 1000000, 32
_VOCAB_B, _DIM_B = 100000, 64
_BATCH, _HIST = 16384, 50

_L = 128
_NW = 32
_BPW = _BATCH // _NW
_KB = _BPW // _L
_BLOCKS = _HIST * _KB
_TR_A = _DIM_A // 8
_TR_B = _DIM_B // 8


def _transpose_block(rows_v, t_v, dim):
    iota = jnp.arange(16, dtype=jnp.int32)

    @plsc.parallel_loop(0, 16, step=1, unroll=8)
    def _krot(k):
        rot = jnp.bitwise_and(iota + k, 15)
        for d0 in range(0, dim, 16):
            d_vec = rot + d0
            tr_vec = jnp.right_shift(d_vec, 3)
            s_vec = jnp.bitwise_and(d_vec, 7)
            for j0 in range(0, _L, 16):
                lanes = iota + j0
                v = plsc.load_gather(rows_v, [lanes, d_vec])
                plsc.store_scatter(t_v, [tr_vec, s_vec, lanes], v)


def _diag_transpose_packed(buf, outb, rows, cols, out_row0):
    iota = jnp.arange(16, dtype=jnp.int32)
    pack = 128 // rows

    @plsc.parallel_loop(0, 16, step=1, unroll=4)
    def _krot(k):
        rot = jnp.bitwise_and(iota + k, 15)
        for d0 in range(0, rows, 16):
            d_vec = rot + d0
            for l0 in range(0, cols, 16):
                lanes = iota + l0
                q_vec = jnp.right_shift(lanes, pack // 2) + out_row0
                z_vec = jnp.bitwise_and(lanes, pack - 1) * rows + d_vec
                v = plsc.load_gather(buf, [d_vec, lanes])
                plsc.store_scatter(outb, [q_vec, z_vec], v)


def _repack_body(ta_hbm, tb_hbm, tail_a_hbm, tail_b_hbm, pa_hbm, pb_hbm,
                 ba0, ba1, oa0, oa1, bb0, bb1, ob0, ob1, sem_g, sem_st):
    nc = plsc.get_sparse_core_info().num_cores
    wid = lax.axis_index("s") * nc + lax.axis_index("c")

    def runp(t_hbm, p_hbm, bufs, outs, rows, cw, npairs):
        rows_out = rows * cw // 128

        def src(q):
            return t_hbm.at[:, pl.ds(q * cw, cw)]

        def dst(q):
            return p_hbm.at[pl.ds(q * rows_out, rows_out)]

        def step(i, carry):
            q0 = jnp.minimum((2 * i) * _NW + wid, npairs - 1)
            q1 = jnp.minimum((2 * i + 1) * _NW + wid, npairs - 1)
            qq = (q0, q1)
            cps = [pltpu.async_copy(src(q0), bufs[0], sem_g),
                   pltpu.async_copy(src(q1), bufs[1], sem_g)]
            for k in range(2):
                cps[k].wait()

                @pl.when(i > 0)
                def _drain():
                    pltpu.make_async_copy(outs[k], dst(qq[k]), sem_st).wait()

                _diag_transpose_packed(bufs[k], outs[k], rows, cw, 0)
                pltpu.async_copy(outs[k], dst(qq[k]), sem_st)
            return carry

        niter = (npairs + 2 * _NW - 1) // (2 * _NW)
        lax.fori_loop(0, niter, step, 0)
        for k in range(2):
            pltpu.make_async_copy(outs[k], dst(jnp.int32(npairs - 1)),
                                  sem_st).wait()

    runp(ta_hbm, pa_hbm, (ba0, ba1), (oa0, oa1), 32, 256, 3906)
    runp(tb_hbm, pb_hbm, (bb0, bb1), (ob0, ob1), 64, 256, 390)

    @pl.when(wid == 0)
    def _tail_a():
        pltpu.sync_copy(tail_a_hbm, oa0.at[pl.ds(0, 16)])
        pltpu.sync_copy(oa0.at[pl.ds(0, 16)], pa_hbm.at[pl.ds(249984, 16)])

    @pl.when(wid == 1)
    def _tail_b():
        pltpu.sync_copy(tb_hbm.at[:, pl.ds(99840, 128)],
                        bb0.at[:, pl.ds(0, 128)])
        _diag_transpose_packed(bb0, ob0, 64, 128, 0)
        pltpu.sync_copy(ob0.at[pl.ds(0, 64)], pb_hbm.at[pl.ds(49920, 64)])

    @pl.when(wid == 2)
    def _tail_b2():
        pltpu.sync_copy(tail_b_hbm, ob1.at[pl.ds(0, 16)])
        pltpu.sync_copy(ob1.at[pl.ds(0, 16)], pb_hbm.at[pl.ds(49984, 16)])


def _body(idx_a_hbm, idx_b_hbm, tab_a_hbm, tab_b_hbm, out_a_hbm, out_b_hbm,
          idxa_v, idxb_v, ra0, ra1, ta0, ta1, rb0, rb1, tb0, tb1,
          sem_g, sem_st):
    nc = plsc.get_sparse_core_info().num_cores
    wid = lax.axis_index("s") * nc + lax.axis_index("c")
    b0 = wid * _BPW

    pltpu.sync_copy(idx_a_hbm.at[:, pl.ds(b0, _BPW)], idxa_v)
    pltpu.sync_copy(idx_b_hbm.at[:, pl.ds(b0, _BPW)], idxb_v)

    def run(tab_hbm, out_hbm, idx_v, rows, ts, dim):
        def dst(g):
            h, kb = g // _KB, g % _KB
            return out_hbm.at[h, :, wid * _KB + kb]

        def gather_copy(g, buf):
            g = jnp.minimum(g, _BLOCKS - 1)
            h, kb = g // _KB, g % _KB
            return pltpu.make_async_copy(
                tab_hbm.at[idx_v.at[h, pl.ds(kb * _L, _L)]], buf, sem_g)

        gather_copy(0, rows[0]).start()
        gather_copy(1, rows[1]).start()

        def step(t, carry):
            g0 = 2 * t
            for k in range(2):
                gather_copy(g0 + k, rows[k]).wait()

                @pl.when(t > 0)
                def _drain():
                    pltpu.make_async_copy(ts[k], dst(g0 + k), sem_st).wait()

                _transpose_block(rows[k], ts[k], dim)
                pltpu.async_copy(ts[k], dst(g0 + k), sem_st)
                gather_copy(g0 + 2 + k, rows[k]).start()
            return carry

        lax.fori_loop(0, _BLOCKS // 2, step, 0)
        for k in range(2):
            gather_copy(_BLOCKS - 1, rows[k]).wait()
            pltpu.make_async_copy(ts[k], dst(_BLOCKS - 2 + k), sem_st).wait()

    run(tab_a_hbm, out_a_hbm, idxa_v, (ra0, ra1), (ta0, ta1), _DIM_A)
    run(tab_b_hbm, out_b_hbm, idxb_v, (rb0, rb1), (tb0, tb1), _DIM_B)


@jax.jit
def _lookup(dom_a_idx, dom_b_idx, table_a, table_b):
    mesh = plsc.VectorSubcoreMesh(core_axis_name="c", subcore_axis_name="s")
    rk = pl.kernel(
        _repack_body,
        out_type=(
            jax.ShapeDtypeStruct((_VOCAB_A * _DIM_A // 128, 128), jnp.float32),
            jax.ShapeDtypeStruct((_VOCAB_B * _DIM_B // 128, 128), jnp.float32),
        ),
        mesh=mesh,
        scratch_types=[
            pltpu.VMEM((32, 256), jnp.float32),
            pltpu.VMEM((32, 256), jnp.float32),
            pltpu.VMEM((64, 128), jnp.float32),
            pltpu.VMEM((64, 128), jnp.float32),
            pltpu.VMEM((64, 256), jnp.float32),
            pltpu.VMEM((64, 256), jnp.float32),
            pltpu.VMEM((128, 128), jnp.float32),
            pltpu.VMEM((128, 128), jnp.float32),
            pltpu.SemaphoreType.DMA,
            pltpu.SemaphoreType.DMA,
        ],
        compiler_params=pltpu.CompilerParams(use_tc_tiling_on_sc=True,
                                             needs_layout_passes=False),
    )
    k = pl.kernel(
        _body,
        out_type=(
            jax.ShapeDtypeStruct((_HIST, _TR_A, _BATCH // _L, 8, _L),
                                 jnp.float32),
            jax.ShapeDtypeStruct((_HIST, _TR_B, _BATCH // _L, 8, _L),
                                 jnp.float32),
        ),
        mesh=mesh,
        scratch_types=[
            pltpu.VMEM((_HIST, _BPW), jnp.int32),
            pltpu.VMEM((_HIST, _BPW), jnp.int32),
            pltpu.VMEM((_L, _DIM_A), jnp.float32),
            pltpu.VMEM((_L, _DIM_A), jnp.float32),
            pltpu.VMEM((_TR_A, 8, _L), jnp.float32),
            pltpu.VMEM((_TR_A, 8, _L), jnp.float32),
            pltpu.VMEM((_L, _DIM_B), jnp.float32),
            pltpu.VMEM((_L, _DIM_B), jnp.float32),
            pltpu.VMEM((_TR_B, 8, _L), jnp.float32),
            pltpu.VMEM((_TR_B, 8, _L), jnp.float32),
            pltpu.SemaphoreType.DMA,
            pltpu.SemaphoreType.DMA,
        ],
        compiler_params=pltpu.CompilerParams(use_tc_tiling_on_sc=False,
                                             needs_layout_passes=False),
    )
    ia = jnp.transpose(dom_a_idx)
    ib = jnp.transpose(dom_b_idx)

    tail_a = table_a[_VOCAB_A - 64:].reshape(16, 128)
    tail_b = table_b[_VOCAB_B - 32:].reshape(16, 128)
    pa, pb = rk(jnp.transpose(table_a), jnp.transpose(table_b),
                tail_a, tail_b)
    ta = pa.reshape(_VOCAB_A, _DIM_A)
    tb = pb.reshape(_VOCAB_B, _DIM_B)

    ka, kb = k(ia, ib, ta, tb)

    def finish(arr, dim):
        p = arr.transpose(0, 1, 3, 2, 4).reshape(_HIST, dim, _BATCH)
        return p.transpose(2, 0, 1)

    return finish(ka, _DIM_A), finish(kb, _DIM_B)


def kernel(dom_a_idx, dom_b_idx, table_a, table_b):
    return _lookup(dom_a_idx, dom_b_idx, table_a, table_b)
